# baseline (speedup 1.0000x reference)
import math

import jax
import jax.numpy as jnp
from jax import lax
from jax.experimental import pallas as pl
from jax.experimental.pallas import tpu as pltpu
from jax.experimental.pallas import tpu_sc as plsc

N = 10000
E = 320000
D = 128
OUT = 128
H = 4
C = OUT // H

NP = 2
E2 = E // NP
NC, NS = 2, 16
NW = NC * NS
EPW = E2 // NW
G = 40
NCH = EPW // G
KB = 5
N2 = 10240
STR = N2 // NS
ZCH = 32
SROW = N2 // 32
SSTR = SROW // 10
EB = 1280
NB = 2000
FB = 2048


def _q_body(x_ref, wq_ref, q_ref):
    q_ref[...] = jnp.dot(x_ref[...], wq_ref[...],
                         preferred_element_type=jnp.float32)


def _make_gather_body(eoff):
    def _gather_body(x_hbm, q_hbm, dst_hbm, src_hbm, xg_hbm, qg_hbm,
                     dsti_v, srci_v, rx_v, rq_v, sg, sw):
        wid = lax.axis_index("s") * NC + lax.axis_index("c")
        base = eoff + wid * EPW
        obase = wid * EPW
        pltpu.sync_copy(dst_hbm.at[pl.ds(base, EPW)], dsti_v)
        pltpu.sync_copy(src_hbm.at[pl.ds(base, EPW)], srci_v)

        def drain_writebacks():
            for b in range(KB):
                pltpu.make_async_copy(x_hbm.at[pl.ds(0, G)], rx_v.at[b],
                                      sw).wait()
                pltpu.make_async_copy(q_hbm.at[pl.ds(0, G)], rq_v.at[b],
                                      sw).wait()

        def group(g, carry):
            @pl.when(g > 0)
            def _():
                drain_writebacks()

            descs = []
            for b in range(KB):
                j = g * KB + b
                descs.append(pltpu.async_copy(
                    x_hbm.at[dsti_v.at[pl.ds(j * G, G)]], rx_v.at[b], sg))
                descs.append(pltpu.async_copy(
                    q_hbm.at[srci_v.at[pl.ds(j * G, G)]], rq_v.at[b], sg))
            for d_ in descs:
                d_.wait()
            for b in range(KB):
                off = obase + (g * KB + b) * G
                pltpu.async_copy(rx_v.at[b], xg_hbm.at[pl.ds(off, G)], sw)
                pltpu.async_copy(rq_v.at[b], qg_hbm.at[pl.ds(off, G)], sw)
            return carry

        lax.fori_loop(0, NCH // KB, group, 0)
        drain_writebacks()

    return _gather_body


def _edge_body(ea_ref, xg_ref, qg_ref, src_ref, wk_ref, wv_ref, we_ref,
               be_ref, s_ref, r_ref, t4_ref, p_ref, p2_ref, eo_ref):
    z = ea_ref[...] * xg_ref[...]
    k = jnp.dot(z, wk_ref[...], preferred_element_type=jnp.float32)
    t = qg_ref[...] * k
    att = jnp.dot(t, s_ref[...], preferred_element_type=jnp.float32)
    ex = jnp.exp(jnp.minimum(att, 80.0))
    exb = jnp.dot(ex, r_ref[...], preferred_element_type=jnp.float32)
    v = jnp.dot(z, wv_ref[...], preferred_element_type=jnp.float32)
    p_ref[...] = v * exb
    ext = jnp.dot(ex, t4_ref[...], preferred_element_type=jnp.float32)
    m32 = jnp.bitwise_and(src_ref[...], 31)
    lane_grp = jax.lax.broadcasted_iota(jnp.int32, (1, OUT), 1) // H
    p2_ref[...] = ext * (m32 == lane_grp).astype(jnp.float32)
    eo_ref[...] = jnp.dot(z, we_ref[...],
                          preferred_element_type=jnp.float32) + be_ref[...]


def _make_scatter_body(eoff):
    def _scatter_body(p_hbm, p2_hbm, src_hbm, src32_hbm, vout_hbm, sout_hbm,
                      idx_v, idx2_v, rows_v, rows2_v, zb_v, acc_sh, acc2_sh,
                      sl0, sl1):
        cid = lax.axis_index("c")
        sid = lax.axis_index("s")
        wid = sid * NC + cid
        zero16 = jnp.zeros((16,), jnp.float32)

        def zb_zero(i, carry):
            zb_v[i // 8, pl.ds((i % 8) * 16, 16)] = zero16
            return carry

        lax.fori_loop(0, ZCH * 8, zb_zero, 0)

        def zcopy(j, carry):
            pltpu.sync_copy(zb_v, acc_sh.at[pl.ds(sid * STR + j * ZCH, ZCH)])
            return carry

        lax.fori_loop(0, STR // ZCH, zcopy, 0)

        @pl.when(sid < 10)
        def _zero2():
            pltpu.sync_copy(zb_v.at[pl.ds(0, SSTR)],
                            acc2_sh.at[pl.ds(sid * SSTR, SSTR)])

        plsc.subcore_barrier()

        base = eoff + wid * EPW
        pbase = wid * EPW

        def load_chunk(j, b, sem):
            off = base + j * G
            poff = pbase + j * G
            pltpu.async_copy(src_hbm.at[pl.ds(off, G)], idx_v.at[b], sem)
            pltpu.async_copy(p_hbm.at[pl.ds(poff, G)], rows_v.at[b], sem)
            pltpu.async_copy(src32_hbm.at[pl.ds(off, G)], idx2_v.at[b], sem)
            pltpu.async_copy(p2_hbm.at[pl.ds(poff, G)], rows2_v.at[b], sem)

        def drain_chunk(b, sem):
            pltpu.make_async_copy(src_hbm.at[pl.ds(0, G)], idx_v.at[b],
                                  sem).wait()
            pltpu.make_async_copy(p_hbm.at[pl.ds(0, G)], rows_v.at[b],
                                  sem).wait()
            pltpu.make_async_copy(src32_hbm.at[pl.ds(0, G)], idx2_v.at[b],
                                  sem).wait()
            pltpu.make_async_copy(p2_hbm.at[pl.ds(0, G)], rows2_v.at[b],
                                  sem).wait()

        load_chunk(0, 0, sl0)

        def body(g, carry):
            b = lax.rem(g, 2)

            @pl.when(b == 0)
            def _even():
                drain_chunk(0, sl0)

                @pl.when(g + 1 < NCH)
                def _():
                    load_chunk(g + 1, 1, sl1)
                pltpu.sync_copy(rows_v.at[0], acc_sh.at[idx_v.at[0]],
                                add=True)
                pltpu.sync_copy(rows2_v.at[0], acc2_sh.at[idx2_v.at[0]],
                                add=True)

            @pl.when(b == 1)
            def _odd():
                drain_chunk(1, sl1)

                @pl.when(g + 1 < NCH)
                def _():
                    load_chunk(g + 1, 0, sl0)
                pltpu.sync_copy(rows_v.at[1], acc_sh.at[idx_v.at[1]],
                                add=True)
                pltpu.sync_copy(rows2_v.at[1], acc2_sh.at[idx2_v.at[1]],
                                add=True)

            return carry

        lax.fori_loop(0, NCH, body, 0)
        plsc.subcore_barrier()

        def dump(j, carry):
            r0 = sid * STR + j * ZCH
            pltpu.sync_copy(acc_sh.at[pl.ds(r0, ZCH)], zb_v)
            pltpu.sync_copy(zb_v, vout_hbm.at[cid, pl.ds(r0, ZCH)])
            return carry

        lax.fori_loop(0, STR // ZCH, dump, 0)

        @pl.when(sid < 10)
        def _dump2():
            r0 = sid * SSTR
            pltpu.sync_copy(acc2_sh.at[pl.ds(r0, SSTR)],
                            zb_v.at[pl.ds(0, SSTR)])
            pltpu.sync_copy(zb_v.at[pl.ds(0, SSTR)],
                            sout_hbm.at[cid, pl.ds(r0, SSTR)])

    return _scatter_body


def _fin_body(va_ref, vb_ref, sa_ref, sb_ref, rsel_ref, out_ref):
    a = va_ref[0] + va_ref[1] + vb_ref[0] + vb_ref[1]
    s4 = sa_ref[0] + sa_ref[1] + sb_ref[0] + sb_ref[1]
    sb = jnp.dot(s4, rsel_ref[...], preferred_element_type=jnp.float32)
    out_ref[...] = a / (sb + 1e-16)


def kernel(x, edge_index, edge_attr, Wq, Wk, Wv, We, be):
    src = edge_index[0, :]
    dst = edge_index[1, :]
    src32 = jax.lax.shift_right_logical(src, 5)
    src2d = src.reshape(E, 1)
    f32 = jnp.float32

    cols = jnp.arange(OUT)
    inv_sqrt_c = 1.0 / math.sqrt(C)
    s_m = ((cols[:, None] // C) == cols[None, :]).astype(f32) * inv_sqrt_c
    r_m = ((cols[:, None]) == (cols[None, :] // C)).astype(f32)
    t4_m = ((cols[:, None]) == (cols[None, :] % H)).astype(f32)
    rsel = (jnp.arange(H)[:, None] == (cols[None, :] // C)).astype(f32)

    q = pl.pallas_call(
        _q_body,
        grid=(N // NB,),
        in_specs=[
            pl.BlockSpec((NB, D), lambda i: (i, 0)),
            pl.BlockSpec((D, OUT), lambda i: (0, 0)),
        ],
        out_specs=pl.BlockSpec((NB, OUT), lambda i: (i, 0)),
        out_shape=jax.ShapeDtypeStruct((N, OUT), f32),
    )(x, Wq)

    mesh = plsc.VectorSubcoreMesh(core_axis_name="c", subcore_axis_name="s")

    def run_gather(eoff):
        g_k = pl.kernel(
            _make_gather_body(eoff),
            out_type=(jax.ShapeDtypeStruct((E2, D), f32),
                      jax.ShapeDtypeStruct((E2, OUT), f32)),
            mesh=mesh,
            scratch_types=[
                pltpu.VMEM((EPW,), jnp.int32),
                pltpu.VMEM((EPW,), jnp.int32),
                pltpu.VMEM((KB, G, D), f32),
                pltpu.VMEM((KB, G, OUT), f32),
                pltpu.SemaphoreType.DMA,
                pltpu.SemaphoreType.DMA,
            ],
        )
        return g_k(x, q, dst, src)

    def run_edge(blkoff, xg, qg):
        ebk = lambda w: pl.BlockSpec((EB, w), lambda i: (i, 0))
        ebko = lambda w: pl.BlockSpec((EB, w),
                                      lambda i, _o=blkoff: (i + _o, 0))
        full = lambda bs: pl.BlockSpec(bs, lambda i: (0, 0))
        return pl.pallas_call(
            _edge_body,
            grid=(E2 // EB,),
            in_specs=[
                ebko(D), ebk(D), ebk(OUT), ebko(1),
                full((D, OUT)), full((D, OUT)), full((D, OUT)),
                full((1, OUT)),
                full((OUT, OUT)), full((OUT, OUT)), full((OUT, OUT)),
            ],
            out_specs=[ebk(OUT), ebk(OUT), ebk(OUT)],
            out_shape=[
                jax.ShapeDtypeStruct((E2, OUT), f32),
                jax.ShapeDtypeStruct((E2, OUT), f32),
                jax.ShapeDtypeStruct((E2, OUT), f32),
            ],
        )(edge_attr, xg, qg, src2d, Wk, Wv, We, be.reshape(1, OUT),
          s_m, r_m, t4_m)

    def run_scatter(eoff, p, p2):
        s_k = pl.kernel(
            _make_scatter_body(eoff),
            out_type=(jax.ShapeDtypeStruct((NC, N2, OUT), f32),
                      jax.ShapeDtypeStruct((NC, SROW, 128), f32)),
            mesh=mesh,
            scratch_types=[
                pltpu.VMEM((2, G), jnp.int32),
                pltpu.VMEM((2, G), jnp.int32),
                pltpu.VMEM((2, G, OUT), f32),
                pltpu.VMEM((2, G, OUT), f32),
                pltpu.VMEM((ZCH, OUT), f32),
                pltpu.VMEM_SHARED((N2, OUT), f32),
                pltpu.VMEM_SHARED((SROW, 128), f32),
                pltpu.SemaphoreType.DMA,
                pltpu.SemaphoreType.DMA,
            ],
        )
        return s_k(p, p2, src, src32)

    xg_a, qg_a = run_gather(0)
    p_a, p2_a, eo_a = run_edge(0, xg_a, qg_a)
    xg_b, qg_b = run_gather(E2)
    p_b, p2_b, eo_b = run_edge(E2 // EB, xg_b, qg_b)
    vacc_a, sacc_a = run_scatter(0, p_a, p2_a)
    vacc_b, sacc_b = run_scatter(E2, p_b, p2_b)

    edge_out = jnp.concatenate([eo_a, eo_b], axis=0)
    s4_a = sacc_a.reshape(NC, SROW * 128 // H, H)
    s4_b = sacc_b.reshape(NC, SROW * 128 // H, H)

    out = pl.pallas_call(
        _fin_body,
        grid=(pl.cdiv(N, FB),),
        in_specs=[
            pl.BlockSpec((NC, FB, OUT), lambda i: (0, i, 0)),
            pl.BlockSpec((NC, FB, OUT), lambda i: (0, i, 0)),
            pl.BlockSpec((NC, FB, H), lambda i: (0, i, 0)),
            pl.BlockSpec((NC, FB, H), lambda i: (0, i, 0)),
            pl.BlockSpec((H, OUT), lambda i: (0, 0)),
        ],
        out_specs=pl.BlockSpec((FB, OUT), lambda i: (i, 0)),
        out_shape=jax.ShapeDtypeStruct((N, OUT), f32),
    )(vacc_a, vacc_b, s4_a, s4_b, rsel)

    return (out, edge_out)


# speedup vs baseline: 1.1056x; 1.1056x over previous
import math

import jax
import jax.numpy as jnp
from jax import lax
from jax.experimental import pallas as pl
from jax.experimental.pallas import tpu as pltpu
from jax.experimental.pallas import tpu_sc as plsc

N = 10000
E = 320000
D = 128
OUT = 128
H = 4
C = OUT // H

NP = 2
E2 = E // NP
NC, NS = 2, 16
NW = NC * NS
EPW = E2 // NW
G = 40
NCH = EPW // G
KB = 5
GS = 80
EPWS = E2 // NS
NCHS = EPWS // GS
N2 = 10240
STR = N2 // NS
ZCH = 64
SROW = N2 // 32
EB = 1280
NB = 2000
FB = 2048


def _q_body(x_ref, wq_ref, q_ref):
    q_ref[...] = jnp.dot(x_ref[...], wq_ref[...],
                         preferred_element_type=jnp.float32)


def _make_gather_body(eoff):
    def _gather_body(x_hbm, q_hbm, dst_hbm, src_hbm, xg_hbm, qg_hbm,
                     dsti_v, srci_v, rx_v, rq_v, sg, sw):
        wid = lax.axis_index("s") * NC + lax.axis_index("c")
        base = eoff + wid * EPW
        obase = wid * EPW
        pltpu.sync_copy(dst_hbm.at[pl.ds(base, EPW)], dsti_v)
        pltpu.sync_copy(src_hbm.at[pl.ds(base, EPW)], srci_v)

        def drain_writebacks():
            for b in range(KB):
                pltpu.make_async_copy(x_hbm.at[pl.ds(0, G)], rx_v.at[b],
                                      sw).wait()
                pltpu.make_async_copy(q_hbm.at[pl.ds(0, G)], rq_v.at[b],
                                      sw).wait()

        def group(g, carry):
            @pl.when(g > 0)
            def _():
                drain_writebacks()

            descs = []
            for b in range(KB):
                j = g * KB + b
                descs.append(pltpu.async_copy(
                    x_hbm.at[dsti_v.at[pl.ds(j * G, G)]], rx_v.at[b], sg))
                descs.append(pltpu.async_copy(
                    q_hbm.at[srci_v.at[pl.ds(j * G, G)]], rq_v.at[b], sg))
            for d_ in descs:
                d_.wait()
            for b in range(KB):
                off = obase + (g * KB + b) * G
                pltpu.async_copy(rx_v.at[b], xg_hbm.at[pl.ds(off, G)], sw)
                pltpu.async_copy(rq_v.at[b], qg_hbm.at[pl.ds(off, G)], sw)
            return carry

        lax.fori_loop(0, NCH // KB, group, 0)
        drain_writebacks()

    return _gather_body


def _edge_core(ea_ref, xg_ref, qg_ref, src_ref, src32t_ref, wk_ref, wv_ref,
               we_ref, be_ref, s_ref, r_ref, t4_ref, p_ref, eo_ref, sacc_ref):
    z = ea_ref[...] * xg_ref[...]
    k = jnp.dot(z, wk_ref[...], preferred_element_type=jnp.float32)
    t = qg_ref[...] * k
    att = jnp.dot(t, s_ref[...], preferred_element_type=jnp.float32)
    ex = jnp.exp(jnp.minimum(att, 80.0))
    exb = jnp.dot(ex, r_ref[...], preferred_element_type=jnp.float32)
    v = jnp.dot(z, wv_ref[...], preferred_element_type=jnp.float32)
    p_ref[...] = v * exb
    eo_ref[...] = jnp.dot(z, we_ref[...],
                          preferred_element_type=jnp.float32) + be_ref[...]
    ext = jnp.dot(ex, t4_ref[...], preferred_element_type=jnp.float32)
    m32 = jnp.bitwise_and(src_ref[...], 31)
    lane_grp = jax.lax.broadcasted_iota(jnp.int32, (1, OUT), 1) // H
    p2row = ext * (m32 == lane_grp).astype(jnp.float32)
    g320 = jax.lax.broadcasted_iota(jnp.int32, (SROW, 1), 0)
    oht = (g320 == src32t_ref[...]).astype(jnp.float32)
    sblk = jnp.dot(oht, p2row, preferred_element_type=jnp.float32)

    @pl.when(pl.program_id(0) == 0)
    def _init():
        sacc_ref[...] = sblk

    @pl.when(pl.program_id(0) > 0)
    def _acc():
        sacc_ref[...] = sacc_ref[...] + sblk


def _edge_body_a(ea_ref, xg_ref, qg_ref, src_ref, src32t_ref, wk_ref, wv_ref,
                 we_ref, be_ref, s_ref, r_ref, t4_ref, p_ref, eo_ref,
                 sacc_ref):
    _edge_core(ea_ref, xg_ref, qg_ref, src_ref, src32t_ref, wk_ref, wv_ref,
               we_ref, be_ref, s_ref, r_ref, t4_ref, p_ref, eo_ref, sacc_ref)


def _edge_body_b(ea_ref, xg_ref, qg_ref, src_ref, src32t_ref, wk_ref, wv_ref,
                 we_ref, be_ref, s_ref, r_ref, t4_ref, eoprev_ref, p_ref,
                 eo_ref, sacc_ref):
    del eoprev_ref
    _edge_core(ea_ref, xg_ref, qg_ref, src_ref, src32t_ref, wk_ref, wv_ref,
               we_ref, be_ref, s_ref, r_ref, t4_ref, p_ref, eo_ref, sacc_ref)


def _scatter_body(pa_hbm, pb_hbm, src_hbm, vout_hbm,
                  idx_v, rows_v, zb_v, acc_sh, sl0, sl1, ss0, ss1, sw):
    cid = lax.axis_index("c")
    sid = lax.axis_index("s")
    wid = sid * NC + cid
    zero16 = jnp.zeros((16,), jnp.float32)

    def zb_zero(i, carry):
        zb_v[0, i // 8, pl.ds((i % 8) * 16, 16)] = zero16
        return carry

    lax.fori_loop(0, ZCH * 8, zb_zero, 0)
    for j in range(STR // ZCH):
        pltpu.async_copy(zb_v.at[0], acc_sh.at[pl.ds(sid * STR + j * ZCH,
                                                     ZCH)], sw)
    for j in range(STR // ZCH):
        pltpu.make_async_copy(acc_sh.at[pl.ds(0, ZCH)], zb_v.at[0],
                              sw).wait()
    plsc.subcore_barrier()

    def run_phase(p_hbm, ibase, pbase):
        def load_chunk(j, b, sem):
            pltpu.async_copy(src_hbm.at[pl.ds(ibase + j * GS, GS)],
                             idx_v.at[b], sem)
            pltpu.async_copy(p_hbm.at[pl.ds(pbase + j * GS, GS)],
                             rows_v.at[b], sem)

        def drain_loads(b, sem):
            pltpu.make_async_copy(src_hbm.at[pl.ds(0, GS)], idx_v.at[b],
                                  sem).wait()
            pltpu.make_async_copy(p_hbm.at[pl.ds(0, GS)], rows_v.at[b],
                                  sem).wait()

        def drain_scatter(b, sem):
            pltpu.make_async_copy(rows_v.at[b], acc_sh.at[pl.ds(0, GS)],
                                  sem).wait()

        load_chunk(0, 0, sl0)

        def body(g, carry):
            b = lax.rem(g, 2)

            @pl.when(b == 0)
            def _even():
                drain_loads(0, sl0)

                @pl.when(g > 0)
                def _():
                    drain_scatter(1, ss1)

                @pl.when(g + 1 < NCHS)
                def _():
                    load_chunk(g + 1, 1, sl1)
                pltpu.async_copy(rows_v.at[0], acc_sh.at[idx_v.at[0]], ss0,
                                 add=True)

            @pl.when(b == 1)
            def _odd():
                drain_loads(1, sl1)
                drain_scatter(0, ss0)

                @pl.when(g + 1 < NCHS)
                def _():
                    load_chunk(g + 1, 0, sl0)
                pltpu.async_copy(rows_v.at[1], acc_sh.at[idx_v.at[1]], ss1,
                                 add=True)

            return carry

        lax.fori_loop(0, NCHS, body, 0)
        drain_scatter((NCHS - 1) % 2, ss0 if (NCHS - 1) % 2 == 0 else ss1)

    @pl.when(wid < NS)
    def _phase_a():
        run_phase(pa_hbm, wid * EPWS, wid * EPWS)

    @pl.when(wid >= NS)
    def _phase_b():
        run_phase(pb_hbm, E2 + (wid - NS) * EPWS, (wid - NS) * EPWS)

    plsc.subcore_barrier()

    def dump(j, carry):
        b = lax.rem(j, 2)
        r0 = sid * STR + j * ZCH

        @pl.when(j >= 2)
        def _():
            pltpu.make_async_copy(vout_hbm.at[cid, pl.ds(0, ZCH)],
                                  zb_v.at[0], sw).wait()

        @pl.when(b == 0)
        def _d0():
            pltpu.sync_copy(acc_sh.at[pl.ds(r0, ZCH)], zb_v.at[0])
            pltpu.async_copy(zb_v.at[0], vout_hbm.at[cid, pl.ds(r0, ZCH)], sw)

        @pl.when(b == 1)
        def _d1():
            pltpu.sync_copy(acc_sh.at[pl.ds(r0, ZCH)], zb_v.at[1])
            pltpu.async_copy(zb_v.at[1], vout_hbm.at[cid, pl.ds(r0, ZCH)], sw)

        return carry

    lax.fori_loop(0, STR // ZCH, dump, 0)
    for _ in range(2):
        pltpu.make_async_copy(vout_hbm.at[cid, pl.ds(0, ZCH)], zb_v.at[0],
                              sw).wait()


def _fin_body(vacc_ref, sa_ref, sb_ref, rsel_ref, out_ref):
    a = vacc_ref[0] + vacc_ref[1]
    s4 = sa_ref[...] + sb_ref[...]
    sb = jnp.dot(s4, rsel_ref[...], preferred_element_type=jnp.float32)
    out_ref[...] = a / (sb + 1e-16)


def kernel(x, edge_index, edge_attr, Wq, Wk, Wv, We, be):
    src = edge_index[0, :]
    dst = edge_index[1, :]
    src2d = src.reshape(E, 1)
    src32t = jax.lax.shift_right_logical(src, 5).reshape(1, E)
    f32 = jnp.float32

    cols = jnp.arange(OUT)
    inv_sqrt_c = 1.0 / math.sqrt(C)
    s_m = ((cols[:, None] // C) == cols[None, :]).astype(f32) * inv_sqrt_c
    r_m = ((cols[:, None]) == (cols[None, :] // C)).astype(f32)
    t4_m = ((cols[:, None]) == (cols[None, :] % H)).astype(f32)
    rsel = (jnp.arange(H)[:, None] == (cols[None, :] // C)).astype(f32)

    q = pl.pallas_call(
        _q_body,
        grid=(N // NB,),
        in_specs=[
            pl.BlockSpec((NB, D), lambda i: (i, 0)),
            pl.BlockSpec((D, OUT), lambda i: (0, 0)),
        ],
        out_specs=pl.BlockSpec((NB, OUT), lambda i: (i, 0)),
        out_shape=jax.ShapeDtypeStruct((N, OUT), f32),
    )(x, Wq)

    mesh = plsc.VectorSubcoreMesh(core_axis_name="c", subcore_axis_name="s")

    def run_gather(eoff):
        g_k = pl.kernel(
            _make_gather_body(eoff),
            out_type=(jax.ShapeDtypeStruct((E2, D), f32),
                      jax.ShapeDtypeStruct((E2, OUT), f32)),
            mesh=mesh,
            scratch_types=[
                pltpu.VMEM((EPW,), jnp.int32),
                pltpu.VMEM((EPW,), jnp.int32),
                pltpu.VMEM((KB, G, D), f32),
                pltpu.VMEM((KB, G, OUT), f32),
                pltpu.SemaphoreType.DMA,
                pltpu.SemaphoreType.DMA,
            ],
        )
        return g_k(x, q, dst, src)

    def run_edge(blkoff, xg, qg, eo_prev):
        ebk = lambda w: pl.BlockSpec((EB, w), lambda i: (i, 0))
        ebko = lambda w: pl.BlockSpec((EB, w),
                                      lambda i, _o=blkoff: (i + _o, 0))
        full = lambda bs: pl.BlockSpec(bs, lambda i: (0, 0))
        in_specs = [
            ebko(D), ebk(D), ebk(OUT), ebko(1),
            pl.BlockSpec((1, EB), lambda i, _o=blkoff: (0, i + _o)),
            full((D, OUT)), full((D, OUT)), full((D, OUT)), full((1, OUT)),
            full((OUT, OUT)), full((OUT, OUT)), full((OUT, OUT)),
        ]
        args = [edge_attr, xg, qg, src2d, src32t, Wk, Wv, We,
                be.reshape(1, OUT), s_m, r_m, t4_m]
        kwargs = {}
        if eo_prev is None:
            body = _edge_body_a
        else:
            body = _edge_body_b
            in_specs.append(pl.BlockSpec((8, OUT), lambda i: (0, 0)))
            args.append(eo_prev)
            kwargs["input_output_aliases"] = {12: 1}
        return pl.pallas_call(
            body,
            grid=(E2 // EB,),
            in_specs=in_specs,
            out_specs=[
                ebk(OUT),
                pl.BlockSpec((EB, OUT), lambda i, _o=blkoff: (i + _o, 0)),
                pl.BlockSpec((SROW, 128), lambda i: (0, 0)),
            ],
            out_shape=[
                jax.ShapeDtypeStruct((E2, OUT), f32),
                jax.ShapeDtypeStruct((E, OUT), f32),
                jax.ShapeDtypeStruct((SROW, 128), f32),
            ],
            **kwargs,
        )(*args)

    xg_a, qg_a = run_gather(0)
    p_a, eo_a, s_a = run_edge(0, xg_a, qg_a, None)
    xg_b, qg_b = run_gather(E2)
    p_b, edge_out, s_b = run_edge(E2 // EB, xg_b, qg_b, eo_a)

    scatter = pl.kernel(
        _scatter_body,
        out_type=jax.ShapeDtypeStruct((NC, N2, OUT), f32),
        mesh=mesh,
        scratch_types=[
            pltpu.VMEM((2, GS), jnp.int32),
            pltpu.VMEM((2, GS, OUT), f32),
            pltpu.VMEM((2, ZCH, OUT), f32),
            pltpu.VMEM_SHARED((N2, OUT), f32),
            pltpu.SemaphoreType.DMA,
            pltpu.SemaphoreType.DMA,
            pltpu.SemaphoreType.DMA,
            pltpu.SemaphoreType.DMA,
            pltpu.SemaphoreType.DMA,
        ],
    )
    vacc = scatter(p_a, p_b, src)

    s4_a = s_a.reshape(SROW * 128 // H, H)
    s4_b = s_b.reshape(SROW * 128 // H, H)

    out = pl.pallas_call(
        _fin_body,
        grid=(pl.cdiv(N, FB),),
        in_specs=[
            pl.BlockSpec((NC, FB, OUT), lambda i: (0, i, 0)),
            pl.BlockSpec((FB, H), lambda i: (i, 0)),
            pl.BlockSpec((FB, H), lambda i: (i, 0)),
            pl.BlockSpec((H, OUT), lambda i: (0, 0)),
        ],
        out_specs=pl.BlockSpec((FB, OUT), lambda i: (i, 0)),
        out_shape=jax.ShapeDtypeStruct((N, OUT), f32),
    )(vacc, s4_a, s4_b, rsel)

    return (out, edge_out)
